# baseline (speedup 1.0000x reference)
import functools

import jax
import jax.numpy as jnp
from jax import lax
from jax.experimental import pallas as pl
from jax.experimental.pallas import tpu as pltpu
from jax.experimental.pallas import tpu_sc as plsc

DIM = 64
BATCH = 4096
HIST = 200

NC = 2
NS = 16
NW = NC * NS
TOTAL = BATCH * HIST
PER_W = TOTAL // NW
G = 128
KSUB = 5
CHUNK = G * KSUB
NCHUNK = PER_W // CHUNK
NIDX = PER_W // G


@functools.partial(
    pl.kernel,
    mesh=plsc.VectorSubcoreMesh(core_axis_name="c", subcore_axis_name="s"),
    out_type=jax.ShapeDtypeStruct((TOTAL, DIM), jnp.float32),
    scratch_types=[
        pltpu.VMEM((NIDX, G), jnp.int32),
        pltpu.VMEM((CHUNK, DIM), jnp.float32),
        pltpu.VMEM((CHUNK, DIM), jnp.float32),
        pltpu.SemaphoreType.DMA,
        pltpu.SemaphoreType.DMA,
        pltpu.SemaphoreType.DMA,
        pltpu.SemaphoreType.DMA,
    ],
    compiler_params=pltpu.CompilerParams(use_tc_tiling_on_sc=False),
)
def _emb_lookup(table_hbm, idx_hbm, out_hbm, idx_v, buf0, buf1, gs0, gs1, ss0, ss1):
    c = lax.axis_index("c")
    s = lax.axis_index("s")
    wid = s * NC + c
    base = wid * PER_W

    pltpu.sync_copy(idx_hbm.at[wid], idx_v)

    bufs = (buf0, buf1)
    gsems = (gs0, gs1)
    ssems = (ss0, ss1)

    def fire_gather(i, b):
        for j in range(KSUB):
            pltpu.async_copy(
                table_hbm.at[idx_v.at[i * KSUB + j]],
                bufs[b].at[pl.ds(j * G, G)],
                gsems[b],
            )

    def wait_gather(b):
        pltpu.make_async_copy(table_hbm.at[pl.ds(0, CHUNK)], bufs[b], gsems[b]).wait()

    def fire_scatter(i, b):
        pltpu.async_copy(bufs[b], out_hbm.at[pl.ds(base + i * CHUNK, CHUNK)], ssems[b])

    def wait_scatter(b):
        pltpu.make_async_copy(table_hbm.at[pl.ds(0, CHUNK)], bufs[b], ssems[b]).wait()

    fire_gather(0, 0)

    def outer(io, _):
        i0 = io * 2
        wait_gather(0)
        fire_scatter(i0, 0)
        pl.when(i0 > 0)(lambda: wait_scatter(1))
        fire_gather(i0 + 1, 1)
        wait_gather(1)
        fire_scatter(i0 + 1, 1)
        wait_scatter(0)
        pl.when(i0 < NCHUNK - 2)(lambda: fire_gather(i0 + 2, 0))
        return _

    lax.fori_loop(0, NCHUNK // 2, outer, None)
    wait_scatter(1)


def kernel(inputs, emb_weight):
    idx = inputs.reshape(NW, NIDX, G)
    out = _emb_lookup(emb_weight, idx)
    return out.reshape(BATCH, HIST, DIM)


# speedup vs baseline: 1.2170x; 1.2170x over previous
import functools

import jax
import jax.numpy as jnp
from jax import lax
from jax.experimental import pallas as pl
from jax.experimental.pallas import tpu as pltpu
from jax.experimental.pallas import tpu_sc as plsc

DIM = 64
PADW = 128
BATCH = 4096
HIST = 200
VOCAB2 = 500000

NC = 2
NS = 16
NW = NC * NS
TOTAL = BATCH * HIST
PER_W = TOTAL // NW
G = 128
KSUB = 2
CHUNK = G * KSUB
NCHUNK = PER_W // CHUNK
NIDX = PER_W // G


@functools.partial(
    pl.kernel,
    mesh=plsc.VectorSubcoreMesh(core_axis_name="c", subcore_axis_name="s"),
    out_type=jax.ShapeDtypeStruct((TOTAL, PADW), jnp.float32),
    scratch_types=[
        pltpu.VMEM((NIDX, G), jnp.int32),
        pltpu.VMEM((CHUNK, PADW), jnp.float32),
        pltpu.VMEM((CHUNK, PADW), jnp.float32),
        pltpu.SemaphoreType.DMA,
        pltpu.SemaphoreType.DMA,
        pltpu.SemaphoreType.DMA,
        pltpu.SemaphoreType.DMA,
    ],
)
def _emb_lookup(table_hbm, idx_hbm, out_hbm, idx_v, buf0, buf1, gs0, gs1, ss0, ss1):
    c = lax.axis_index("c")
    s = lax.axis_index("s")
    wid = s * NC + c
    base = wid * PER_W

    pltpu.sync_copy(idx_hbm.at[wid], idx_v)

    bufs = (buf0, buf1)
    gsems = (gs0, gs1)
    ssems = (ss0, ss1)

    def fire_gather(i, b):
        for j in range(KSUB):
            pltpu.async_copy(
                table_hbm.at[idx_v.at[i * KSUB + j]],
                bufs[b].at[pl.ds(j * G, G)],
                gsems[b],
            )

    def wait_gather(b):
        pltpu.make_async_copy(table_hbm.at[pl.ds(0, CHUNK)], bufs[b], gsems[b]).wait()

    def fire_scatter(i, b):
        pltpu.async_copy(
            bufs[b],
            out_hbm.at[pl.ds(base + i * CHUNK, CHUNK)],
            ssems[b],
        )

    def wait_scatter(b):
        pltpu.make_async_copy(
            out_hbm.at[pl.ds(0, CHUNK)], bufs[b], ssems[b]
        ).wait()

    fire_gather(0, 0)

    def outer(io, _):
        i0 = io * 2
        wait_gather(0)
        fire_scatter(i0, 0)
        pl.when(i0 > 0)(lambda: wait_scatter(1))
        fire_gather(i0 + 1, 1)
        wait_gather(1)
        fire_scatter(i0 + 1, 1)
        wait_scatter(0)
        pl.when(i0 < NCHUNK - 2)(lambda: fire_gather(i0 + 2, 0))
        return _

    lax.fori_loop(0, NCHUNK // 2, outer, None)
    wait_scatter(1)


def kernel(inputs, emb_weight):
    table = jnp.pad(emb_weight, ((0, 0), (0, PADW - DIM)))
    idx = inputs.reshape(NW, NIDX, G)
    out = _emb_lookup(table, idx)
    return out[:, :DIM].reshape(BATCH, HIST, DIM)


# speedup vs baseline: 1.2184x; 1.0012x over previous
import functools

import jax
import jax.numpy as jnp
from jax import lax
from jax.experimental import pallas as pl
from jax.experimental.pallas import tpu as pltpu
from jax.experimental.pallas import tpu_sc as plsc

DIM = 64
PADW = 128
BATCH = 4096
HIST = 200
VOCAB2 = 500000

NC = 2
NS = 16
NW = NC * NS
TOTAL = BATCH * HIST
PER_W = TOTAL // NW
G = 128
KSUB = 2
CHUNK = G * KSUB
NCHUNK = PER_W // CHUNK
NIDX = PER_W // G


@functools.partial(
    pl.kernel,
    mesh=plsc.VectorSubcoreMesh(core_axis_name="c", subcore_axis_name="s"),
    out_type=jax.ShapeDtypeStruct((TOTAL, PADW), jnp.float32),
    scratch_types=[
        pltpu.VMEM((NIDX, G), jnp.int32),
        pltpu.VMEM((CHUNK, PADW), jnp.float32),
        pltpu.VMEM((CHUNK, PADW), jnp.float32),
        pltpu.SemaphoreType.DMA,
        pltpu.SemaphoreType.DMA,
        pltpu.SemaphoreType.DMA,
        pltpu.SemaphoreType.DMA,
    ],
    compiler_params=pltpu.CompilerParams(skip_device_barrier=True),
)
def _emb_lookup(table_hbm, idx_hbm, out_hbm, idx_v, buf0, buf1, gs0, gs1, ss0, ss1):
    c = lax.axis_index("c")
    s = lax.axis_index("s")
    wid = s * NC + c
    base = wid * PER_W

    pltpu.sync_copy(idx_hbm.at[wid], idx_v)

    bufs = (buf0, buf1)
    gsems = (gs0, gs1)
    ssems = (ss0, ss1)

    def fire_gather(i, b):
        for j in range(KSUB):
            pltpu.async_copy(
                table_hbm.at[idx_v.at[i * KSUB + j]],
                bufs[b].at[pl.ds(j * G, G)],
                gsems[b],
            )

    def wait_gather(b):
        pltpu.make_async_copy(table_hbm.at[pl.ds(0, CHUNK)], bufs[b], gsems[b]).wait()

    def fire_scatter(i, b):
        pltpu.async_copy(
            bufs[b],
            out_hbm.at[pl.ds(base + i * CHUNK, CHUNK)],
            ssems[b],
        )

    def wait_scatter(b):
        pltpu.make_async_copy(
            out_hbm.at[pl.ds(0, CHUNK)], bufs[b], ssems[b]
        ).wait()

    fire_gather(0, 0)

    def outer(io, _):
        i0 = io * 2
        wait_gather(0)
        fire_scatter(i0, 0)
        pl.when(i0 > 0)(lambda: wait_scatter(1))
        fire_gather(i0 + 1, 1)
        wait_gather(1)
        fire_scatter(i0 + 1, 1)
        wait_scatter(0)
        pl.when(i0 < NCHUNK - 2)(lambda: fire_gather(i0 + 2, 0))
        return _

    lax.fori_loop(0, NCHUNK // 2, outer, None)
    wait_scatter(1)


def kernel(inputs, emb_weight):
    table = jnp.pad(emb_weight, ((0, 0), (0, PADW - DIM)))
    idx = inputs.reshape(NW, NIDX, G)
    out = _emb_lookup(table, idx)
    return out[:, :DIM].reshape(BATCH, HIST, DIM)


# speedup vs baseline: 1.2983x; 1.0655x over previous
import functools

import jax
import jax.numpy as jnp
from jax import lax
from jax.experimental import pallas as pl
from jax.experimental.pallas import tpu as pltpu
from jax.experimental.pallas import tpu_sc as plsc

DIM = 64
PADW = 128
BATCH = 4096
HIST = 200
VOCAB2 = 500000

NC = 2
NS = 16
NW = NC * NS
TOTAL = BATCH * HIST
PER_W = TOTAL // NW
G = 128
KSUB = 2
CHUNK = G * KSUB
NCHUNK = PER_W // CHUNK
NIDX = PER_W // G


TBLK = 2048
NTBLK = (VOCAB2 * 2 + TBLK - 1) // TBLK


@functools.partial(
    pl.pallas_call,
    grid=(NTBLK,),
    in_specs=[pl.BlockSpec((DIM, TBLK), lambda j: (0, j))],
    out_specs=pl.BlockSpec((TBLK, PADW), lambda j: (j, 0)),
    out_shape=jax.ShapeDtypeStruct((VOCAB2 * 2, PADW), jnp.float32),
)
def _transpose_pad(tT_ref, out_ref):
    t = tT_ref[...].T
    out_ref[...] = jnp.concatenate([t, jnp.zeros_like(t)], axis=1)


@functools.partial(
    pl.kernel,
    mesh=plsc.VectorSubcoreMesh(core_axis_name="c", subcore_axis_name="s"),
    out_type=jax.ShapeDtypeStruct((TOTAL, PADW), jnp.float32),
    scratch_types=[
        pltpu.VMEM((NIDX, G), jnp.int32),
        pltpu.VMEM((CHUNK, PADW), jnp.float32),
        pltpu.VMEM((CHUNK, PADW), jnp.float32),
        pltpu.SemaphoreType.DMA,
        pltpu.SemaphoreType.DMA,
        pltpu.SemaphoreType.DMA,
        pltpu.SemaphoreType.DMA,
    ],
    compiler_params=pltpu.CompilerParams(skip_device_barrier=True),
)
def _emb_lookup(table_hbm, idx_hbm, out_hbm, idx_v, buf0, buf1, gs0, gs1, ss0, ss1):
    c = lax.axis_index("c")
    s = lax.axis_index("s")
    wid = s * NC + c
    base = wid * PER_W

    pltpu.sync_copy(idx_hbm.at[wid], idx_v)

    bufs = (buf0, buf1)
    gsems = (gs0, gs1)
    ssems = (ss0, ss1)

    def fire_gather(i, b):
        for j in range(KSUB):
            pltpu.async_copy(
                table_hbm.at[idx_v.at[i * KSUB + j]],
                bufs[b].at[pl.ds(j * G, G)],
                gsems[b],
            )

    def wait_gather(b):
        pltpu.make_async_copy(table_hbm.at[pl.ds(0, CHUNK)], bufs[b], gsems[b]).wait()

    def fire_scatter(i, b):
        pltpu.async_copy(
            bufs[b],
            out_hbm.at[pl.ds(base + i * CHUNK, CHUNK)],
            ssems[b],
        )

    def wait_scatter(b):
        pltpu.make_async_copy(
            out_hbm.at[pl.ds(0, CHUNK)], bufs[b], ssems[b]
        ).wait()

    fire_gather(0, 0)

    def outer(io, _):
        i0 = io * 2
        wait_gather(0)
        fire_scatter(i0, 0)
        pl.when(i0 > 0)(lambda: wait_scatter(1))
        fire_gather(i0 + 1, 1)
        wait_gather(1)
        fire_scatter(i0 + 1, 1)
        wait_scatter(0)
        pl.when(i0 < NCHUNK - 2)(lambda: fire_gather(i0 + 2, 0))
        return _

    lax.fori_loop(0, NCHUNK // 2, outer, None)
    wait_scatter(1)


def kernel(inputs, emb_weight):
    table = _transpose_pad(emb_weight.T)
    idx = inputs.reshape(NW, NIDX, G)
    out = _emb_lookup(table, idx)
    return out[:, :DIM].reshape(BATCH, HIST, DIM)
